# Initial kernel scaffold; baseline (speedup 1.0000x reference)
import jax
import jax.numpy as jnp
from jax.experimental import pallas as pl


def kernel(node_embeddings, edge_relations, adjacency, W_msg, b_msg, W_up, b_up, bn_gamma, bn_beta):
    raise NotImplementedError("write your pallas kernel here")


# speedup vs baseline: 13.8000x; 13.8000x over previous
import jax
import jax.numpy as jnp
from jax import lax
from jax.experimental import pallas as pl

B, N, H = 8, 128, 128


def _fused_kernel(adj_ref, adjt_ref, ne_ref, ert_ref, wnT_ref, weT_ref,
                  bmsg_ref, wu1T_ref, wu2T_ref, bup_ref, gamma_ref, beta_ref,
                  out_ref):
    f32 = jnp.float32
    mask = (adj_ref[:] > 0).astype(f32)
    maskT = (adjt_ref[:] > 0).astype(f32)
    deg = jnp.sum(mask, axis=1, keepdims=True)
    bias_jh = deg * bmsg_ref[:]
    weT = weT_ref[:]
    wnT = wnT_ref[:]
    wu1T = wu1T_ref[:]
    wu2T = wu2T_ref[:]
    bup = bup_ref[:]

    s = jnp.zeros((1, H), f32)
    s2 = jnp.zeros((1, H), f32)
    for b in range(B):
        ne_b = ne_ref[b]
        red_b = jnp.sum(ert_ref[b] * maskT[None, :, :], axis=1)
        term_b = lax.dot_general(red_b, weT, (((0,), (0,)), ((), ())),
                                 preferred_element_type=f32)
        proj_b = jnp.dot(ne_b, wnT, preferred_element_type=f32)
        msg_b = jnp.dot(mask, proj_b, preferred_element_type=f32) \
            + term_b + bias_jh
        up_b = jnp.dot(ne_b, wu1T, preferred_element_type=f32) \
            + jnp.dot(msg_b, wu2T, preferred_element_type=f32) + bup
        up_b = jnp.maximum(up_b, 0.0)
        out_ref[b] = up_b
        s = s + jnp.sum(up_b, axis=0, keepdims=True)
        s2 = s2 + jnp.sum(up_b * up_b, axis=0, keepdims=True)

    inv_n = 1.0 / (B * N)
    mean = s * inv_n
    var = s2 * inv_n - mean * mean
    scale = lax.rsqrt(var + 1e-5) * gamma_ref[:]
    shift = beta_ref[:] - mean * scale
    for b in range(B):
        out_ref[b] = out_ref[b] * scale + shift + ne_ref[b]


def kernel(node_embeddings, edge_relations, adjacency, W_msg, b_msg,
           W_up, b_up, bn_gamma, bn_beta):
    ne = node_embeddings.astype(jnp.float32)
    adj = adjacency.astype(jnp.int32)
    adjt = adj.T
    ert = jnp.transpose(edge_relations, (0, 3, 1, 2))
    wnT = W_msg[:, :H].T
    weT = W_msg[:, H:].T
    wu1T = W_up[:, :H].T
    wu2T = W_up[:, H:].T
    bmsg = b_msg.reshape(1, H)
    bup = b_up.reshape(1, H)
    gamma = bn_gamma.reshape(1, H)
    beta = bn_beta.reshape(1, H)
    return pl.pallas_call(
        _fused_kernel,
        out_shape=jax.ShapeDtypeStruct((B, N, H), jnp.float32),
    )(adj, adjt, ne, ert, wnT, weT, bmsg, wu1T, wu2T, bup, gamma, beta)
